# baseline (speedup 1.0000x reference)
import functools

import jax
import jax.numpy as jnp
from jax import lax
from jax.experimental import pallas as pl
from jax.experimental.pallas import tpu as pltpu
from jax.experimental.pallas import tpu_sc as plsc

SEQ = 200
BATCH = 4096
HIDDEN = 64
WIDE = 2 * HIDDEN
VOCAB = 1000000
CHUNK = 128
NC = 2
NS = 16
NW = NC * NS
CPW = SEQ
NBUF = 4
LOOK = 2


def _gather_body(idx_hbm, table_hbm, out_hbm, idx_v, sk, *rest):
    gbufs = rest[:NBUF]
    obufs = rest[NBUF:NBUF + 2]
    sems = rest[NBUF + 2:2 * NBUF + 2]
    stsems = rest[2 * NBUF + 2:]
    wid = lax.axis_index("s") * NC + lax.axis_index("c")
    col0 = wid * CHUNK
    lanes = jnp.arange(16, dtype=jnp.int32)

    def out_at(c):
        return out_hbm.at[c, :, pl.ds(col0, CHUNK)]

    def gather(c, b):
        pltpu.make_async_copy(
            table_hbm.at[idx_v.at[c]], gbufs[b], sems[b]).start()

    def store(c, tb):
        return pltpu.make_async_copy(obufs[tb], out_at(c), stsems[tb])

    def transpose(b, tb):
        gb, ob = gbufs[b], obufs[tb]

        def skew(r, carry):
            rv = jnp.full((16,), r, dtype=jnp.int32)
            for j in range(4):
                cv = (j * 16 + lanes + r) & (HIDDEN - 1)
                plsc.store_scatter(sk, [rv, cv], gb[r, pl.ds(j * 16, 16)])
            return carry

        lax.fori_loop(0, CHUNK, skew, 0)

        def unskew(h, carry):
            for j in range(8):
                rv = j * 16 + lanes
                cv = (h + rv) & (HIDDEN - 1)
                ob[h, pl.ds(j * 16, 16)] = plsc.load_gather(sk, [rv, cv])
            return carry

        lax.fori_loop(0, HIDDEN, unskew, 0)

    pltpu.sync_copy(idx_hbm.at[:, pl.ds(col0, CHUNK)], idx_v)

    for c in range(LOOK):
        gather(c, c % NBUF)

    def group(g, carry):
        for b in range(NBUF):
            c = g * NBUF + b
            pb = (b + LOOK) % NBUF
            tb = b % 2

            @pl.when(c + LOOK < CPW)
            def _():
                gather(c + LOOK, pb)

            pltpu.make_async_copy(
                table_hbm.at[idx_v.at[c]], gbufs[b], sems[b]).wait()

            @pl.when(c >= 2)
            def _():
                store(c - 2, tb).wait()

            transpose(b, tb)
            store(c, tb).start()
        return carry

    lax.fori_loop(0, CPW // NBUF, group, 0)

    for c in range(CPW - 2, CPW):
        store(c, c % 2).wait()


def kernel(input_ids, table):
    mesh = plsc.VectorSubcoreMesh(core_axis_name="c", subcore_axis_name="s")
    t2 = jnp.pad(table, ((0, 0), (0, WIDE - HIDDEN)))
    run = functools.partial(
        pl.kernel,
        mesh=mesh,
        compiler_params=pltpu.CompilerParams(
            use_tc_tiling_on_sc=False, needs_layout_passes=False),
        out_type=jax.ShapeDtypeStruct((SEQ, HIDDEN, BATCH), jnp.float32),
        scratch_types=[pltpu.VMEM((CPW, CHUNK), jnp.int32),
                       pltpu.VMEM((CHUNK, HIDDEN), jnp.float32)]
        + [pltpu.VMEM((CHUNK, WIDE), jnp.float32) for _ in range(NBUF)]
        + [pltpu.VMEM((HIDDEN, CHUNK), jnp.float32) for _ in range(2)]
        + [pltpu.SemaphoreType.DMA for _ in range(NBUF + 2)],
    )(_gather_body)
    outT = run(input_ids.astype(jnp.int32), t2)
    return jnp.transpose(outT, (0, 2, 1))


# speedup vs baseline: 1.5756x; 1.5756x over previous
import functools

import jax
import jax.numpy as jnp
from jax import lax
from jax.experimental import pallas as pl
from jax.experimental.pallas import tpu as pltpu
from jax.experimental.pallas import tpu_sc as plsc

SEQ = 200
BATCH = 4096
HIDDEN = 64
WIDE = 2 * HIDDEN
VOCAB = 1000000
TOT = SEQ * BATCH
CHUNK = 128
NC = 2
NS = 16
NW = NC * NS
CPW = SEQ
NBUF = 4
LOOK = 2


def _gather_body(idx_hbm, t2_hbm, out_hbm, idx_v, *rest):
    gbufs = rest[:NBUF]
    cbufs = rest[NBUF:NBUF + 2]
    sems = rest[NBUF + 2:2 * NBUF + 2]
    stsems = rest[2 * NBUF + 2:]
    wid = lax.axis_index("s") * NC + lax.axis_index("c")
    col0 = wid * CHUNK

    def out_at(c):
        return out_hbm.at[pl.ds(c * BATCH + col0, CHUNK)]

    def gather(c, b):
        pltpu.make_async_copy(t2_hbm.at[idx_v.at[c]], gbufs[b], sems[b]).start()

    def store(c, cb):
        return pltpu.make_async_copy(cbufs[cb], out_at(c), stsems[cb])

    def compact(b, cb):
        gb, ob = gbufs[b], cbufs[cb]

        def rows(r4, carry):
            for rr in range(4):
                r = r4 * 4 + rr
                for j in range(4):
                    ob[r, pl.ds(j * 16, 16)] = gb[r, pl.ds(j * 16, 16)]
            return carry

        lax.fori_loop(0, CHUNK // 4, rows, 0)

    pltpu.sync_copy(idx_hbm.at[:, pl.ds(col0, CHUNK)], idx_v)

    for c in range(LOOK):
        gather(c, c % NBUF)

    def group(g, carry):
        for b in range(NBUF):
            c = g * NBUF + b
            pb = (b + LOOK) % NBUF
            cb = b % 2

            @pl.when(c + LOOK < CPW)
            def _():
                gather(c + LOOK, pb)

            pltpu.make_async_copy(t2_hbm.at[idx_v.at[c]], gbufs[b], sems[b]).wait()

            @pl.when(c >= 2)
            def _():
                store(c - 2, cb).wait()

            compact(b, cb)
            store(c, cb).start()
        return carry

    lax.fori_loop(0, CPW // NBUF, group, 0)

    for c in range(CPW - 2, CPW):
        store(c, c % 2).wait()


def kernel(input_ids, table):
    mesh = plsc.VectorSubcoreMesh(core_axis_name="c", subcore_axis_name="s")
    t2 = jnp.pad(table, ((0, 0), (0, WIDE - HIDDEN)))
    run = functools.partial(
        pl.kernel,
        mesh=mesh,
        out_type=jax.ShapeDtypeStruct((TOT, HIDDEN), jnp.float32),
        scratch_types=[pltpu.VMEM((CPW, CHUNK), jnp.int32)]
        + [pltpu.VMEM((CHUNK, WIDE), jnp.float32) for _ in range(NBUF)]
        + [pltpu.VMEM((CHUNK, HIDDEN), jnp.float32) for _ in range(2)]
        + [pltpu.SemaphoreType.DMA for _ in range(NBUF + 2)],
    )(_gather_body)
    out = run(input_ids.astype(jnp.int32), t2)
    return out.reshape(SEQ, BATCH, HIDDEN)
